# Initial kernel scaffold; baseline (speedup 1.0000x reference)
import jax
import jax.numpy as jnp
from jax.experimental import pallas as pl


def kernel(embs, ngram_embs, table, W_i2h, b_i2h, W_h2o, b_h2o):
    raise NotImplementedError("write your pallas kernel here")


# speedup vs baseline: 1.1110x; 1.1110x over previous
import functools

import jax
import jax.numpy as jnp
from jax import lax
from jax.experimental import pallas as pl
from jax.experimental.pallas import tpu as pltpu
from jax.experimental.pallas import tpu_sc as plsc

B = 4096
D = 128
NG = 200
WL = 50
H = 100
C = 10

NC = 2
NS = 16
NW = NC * NS
B_PER_W = B // NW
LANES = 16

_mesh = plsc.VectorSubcoreMesh(core_axis_name="c", subcore_axis_name="s")


@functools.partial(
    pl.kernel,
    out_type=jax.ShapeDtypeStruct((B, D), jnp.float32),
    mesh=_mesh,
    scratch_types=[
        pltpu.VMEM((B_PER_W * NG,), jnp.int32),
        pltpu.VMEM((NG, D), jnp.float32),
        pltpu.VMEM((B_PER_W, D), jnp.float32),
        pltpu.SemaphoreType.DMA,
    ],
)
def _ngram_sum(idx_hbm, table_hbm, out_hbm, idx_v, rows_v, out_v, sem):
    wid = lax.axis_index("s") * NC + lax.axis_index("c")
    base = pl.multiple_of(wid * B_PER_W, B_PER_W)
    pltpu.sync_copy(idx_hbm.at[pl.ds(base * NG, B_PER_W * NG)], idx_v)

    def row_body(r, _):
        off = pl.multiple_of(r * NG, 8)
        c1 = pltpu.async_copy(
            table_hbm.at[idx_v.at[pl.ds(off, 128)]],
            rows_v.at[pl.ds(0, 128)], sem)
        c2 = pltpu.async_copy(
            table_hbm.at[idx_v.at[pl.ds(off + 128, NG - 128)]],
            rows_v.at[pl.ds(128, NG - 128)], sem)
        c1.wait()
        c2.wait()

        def accum(j, accs):
            return tuple(
                accs[d] + rows_v[j, pl.ds(d * LANES, LANES)]
                for d in range(D // LANES)
            )

        accs = lax.fori_loop(
            0, NG, accum,
            tuple(jnp.zeros((LANES,), jnp.float32) for _ in range(D // LANES)))
        for d in range(D // LANES):
            out_v[r, pl.ds(d * LANES, LANES)] = accs[d]
        return 0

    lax.fori_loop(0, B_PER_W, row_body, 0)
    pltpu.sync_copy(out_v, out_hbm.at[pl.ds(base, B_PER_W)])


BB = 256


def _head_body(embs_ref, ng_ref, w1_ref, b1_ref, w2_ref, b2_ref, o_ref):
    s = jnp.sum(embs_ref[...], axis=1) + ng_ref[...]
    x = s * (1.0 / (WL + NG))
    h = lax.dot_general(x, w1_ref[...], (((1,), (1,)), ((), ())),
                        preferred_element_type=jnp.float32) + b1_ref[...]
    logits = lax.dot_general(h, w2_ref[...], (((1,), (1,)), ((), ())),
                             preferred_element_type=jnp.float32) + b2_ref[...]
    o_ref[...] = jax.nn.sigmoid(logits)


_head = pl.pallas_call(
    _head_body,
    grid=(B // BB,),
    in_specs=[
        pl.BlockSpec((BB, WL, D), lambda i: (i, 0, 0)),
        pl.BlockSpec((BB, D), lambda i: (i, 0)),
        pl.BlockSpec((H, D), lambda i: (0, 0)),
        pl.BlockSpec((1, H), lambda i: (0, 0)),
        pl.BlockSpec((C, H), lambda i: (0, 0)),
        pl.BlockSpec((1, C), lambda i: (0, 0)),
    ],
    out_specs=pl.BlockSpec((BB, C), lambda i: (i, 0)),
    out_shape=jax.ShapeDtypeStruct((B, C), jnp.float32),
)


def kernel(embs, ngram_embs, table, W_i2h, b_i2h, W_h2o, b_h2o):
    idx = ngram_embs.astype(jnp.int32).reshape(-1)
    ng_sum = _ngram_sum(idx, table)
    return _head(embs, ng_sum, W_i2h, b_i2h.reshape(1, H),
                 W_h2o, b_h2o.reshape(1, C))


# speedup vs baseline: 1.6084x; 1.4477x over previous
import functools

import jax
import jax.numpy as jnp
from jax import lax
from jax.experimental import pallas as pl
from jax.experimental.pallas import tpu as pltpu
from jax.experimental.pallas import tpu_sc as plsc

B = 4096
D = 128
NG = 200
WL = 50
H = 100
C = 10

NC = 2
NS = 16
NW = NC * NS
B_PER_W = B // NW
LANES = 16

_mesh = plsc.VectorSubcoreMesh(core_axis_name="c", subcore_axis_name="s")


@functools.partial(
    pl.kernel,
    out_type=jax.ShapeDtypeStruct((B, D), jnp.float32),
    mesh=_mesh,
    scratch_types=[
        pltpu.VMEM((B_PER_W * NG,), jnp.int32),
        pltpu.VMEM((2, NG, D), jnp.float32),
        pltpu.VMEM((B_PER_W, D), jnp.float32),
        pltpu.SemaphoreType.DMA,
        pltpu.SemaphoreType.DMA,
    ],
)
def _ngram_sum(idx_hbm, table_hbm, out_hbm, idx_v, rows_v, out_v, sem0, sem1):
    wid = lax.axis_index("s") * NC + lax.axis_index("c")
    base = pl.multiple_of(wid * B_PER_W, B_PER_W)
    pltpu.sync_copy(idx_hbm.at[pl.ds(base * NG, B_PER_W * NG)], idx_v)

    sems = (sem0, sem1)

    def fire(r, buf):
        off = pl.multiple_of(r * NG, 8)
        pltpu.async_copy(
            table_hbm.at[idx_v.at[pl.ds(off, 128)]],
            rows_v.at[buf].at[pl.ds(0, 128)], sems[buf])
        pltpu.async_copy(
            table_hbm.at[idx_v.at[pl.ds(off + 128, NG - 128)]],
            rows_v.at[buf].at[pl.ds(128, NG - 128)], sems[buf])

    def drain(buf):
        pltpu.make_async_copy(
            table_hbm.at[pl.ds(0, NG)], rows_v.at[buf], sems[buf]).wait()

    def accumulate(buf, r):
        def accum(j, accs):
            a = tuple(
                accs[d] + rows_v[buf, 2 * j, pl.ds(d * LANES, LANES)]
                for d in range(D // LANES))
            return tuple(
                a[d] + rows_v[buf, 2 * j + 1, pl.ds(d * LANES, LANES)]
                for d in range(D // LANES))

        accs = lax.fori_loop(
            0, NG // 2, accum,
            tuple(jnp.zeros((LANES,), jnp.float32) for _ in range(D // LANES)))
        for d in range(D // LANES):
            out_v[r, pl.ds(d * LANES, LANES)] = accs[d]

    fire(0, 0)

    def pair_body(rr, _):
        r0 = 2 * rr
        fire(r0 + 1, 1)
        drain(0)
        accumulate(0, r0)

        @pl.when(rr < B_PER_W // 2 - 1)
        def _():
            fire(r0 + 2, 0)

        drain(1)
        accumulate(1, r0 + 1)
        return 0

    lax.fori_loop(0, B_PER_W // 2, pair_body, 0)
    pltpu.sync_copy(out_v, out_hbm.at[pl.ds(base, B_PER_W)])


BB = 256


def _head_body(embs_ref, ng_ref, w1_ref, b1_ref, w2_ref, b2_ref, o_ref):
    s = jnp.sum(embs_ref[...], axis=1) + ng_ref[...]
    x = s * (1.0 / (WL + NG))
    h = lax.dot_general(x, w1_ref[...], (((1,), (1,)), ((), ())),
                        preferred_element_type=jnp.float32) + b1_ref[...]
    logits = lax.dot_general(h, w2_ref[...], (((1,), (1,)), ((), ())),
                             preferred_element_type=jnp.float32) + b2_ref[...]
    o_ref[...] = jax.nn.sigmoid(logits)


_head = pl.pallas_call(
    _head_body,
    grid=(B // BB,),
    in_specs=[
        pl.BlockSpec((BB, WL, D), lambda i: (i, 0, 0)),
        pl.BlockSpec((BB, D), lambda i: (i, 0)),
        pl.BlockSpec((H, D), lambda i: (0, 0)),
        pl.BlockSpec((1, H), lambda i: (0, 0)),
        pl.BlockSpec((C, H), lambda i: (0, 0)),
        pl.BlockSpec((1, C), lambda i: (0, 0)),
    ],
    out_specs=pl.BlockSpec((BB, C), lambda i: (i, 0)),
    out_shape=jax.ShapeDtypeStruct((B, C), jnp.float32),
)


def kernel(embs, ngram_embs, table, W_i2h, b_i2h, W_h2o, b_h2o):
    idx = ngram_embs.astype(jnp.int32).reshape(-1)
    ng_sum = _ngram_sum(idx, table)
    return _head(embs, ng_sum, W_i2h, b_i2h.reshape(1, H),
                 W_h2o, b_h2o.reshape(1, C))


# speedup vs baseline: 1.6246x; 1.0101x over previous
import functools

import jax
import jax.numpy as jnp
from jax import lax
from jax.experimental import pallas as pl
from jax.experimental.pallas import tpu as pltpu
from jax.experimental.pallas import tpu_sc as plsc

B = 4096
D = 128
NG = 200
WL = 50
H = 100
C = 10

NC = 2
NS = 16
NW = NC * NS
B_PER_W = B // NW
LANES = 16

_mesh = plsc.VectorSubcoreMesh(core_axis_name="c", subcore_axis_name="s")


@functools.partial(
    pl.kernel,
    out_type=jax.ShapeDtypeStruct((B, D), jnp.float32),
    mesh=_mesh,
    scratch_types=[
        pltpu.VMEM((B_PER_W * NG,), jnp.int32),
        pltpu.VMEM((2, NG, D), jnp.float32),
        pltpu.VMEM((B_PER_W, D), jnp.float32),
        pltpu.SemaphoreType.DMA,
        pltpu.SemaphoreType.DMA,
    ],
)
def _ngram_sum(idx_hbm, table_hbm, out_hbm, idx_v, rows_v, out_v, sem0, sem1):
    wid = lax.axis_index("s") * NC + lax.axis_index("c")
    base = pl.multiple_of(wid * B_PER_W, B_PER_W)
    pltpu.sync_copy(idx_hbm.at[pl.ds(base * NG, B_PER_W * NG)], idx_v)

    sems = (sem0, sem1)

    def fire(r, buf):
        off = pl.multiple_of(r * NG, 8)
        pltpu.async_copy(
            table_hbm.at[idx_v.at[pl.ds(off, 128)]],
            rows_v.at[buf].at[pl.ds(0, 128)], sems[buf])
        pltpu.async_copy(
            table_hbm.at[idx_v.at[pl.ds(off + 128, NG - 128)]],
            rows_v.at[buf].at[pl.ds(128, NG - 128)], sems[buf])

    def drain(buf):
        pltpu.make_async_copy(
            table_hbm.at[pl.ds(0, NG)], rows_v.at[buf], sems[buf]).wait()

    def accumulate(buf, r):
        def accum(j, accs):
            a = tuple(
                accs[d] + rows_v[buf, 2 * j, pl.ds(d * LANES, LANES)]
                for d in range(D // LANES))
            return tuple(
                a[d] + rows_v[buf, 2 * j + 1, pl.ds(d * LANES, LANES)]
                for d in range(D // LANES))

        accs = lax.fori_loop(
            0, NG // 2, accum,
            tuple(jnp.zeros((LANES,), jnp.float32) for _ in range(D // LANES)))
        for d in range(D // LANES):
            out_v[r, pl.ds(d * LANES, LANES)] = accs[d]

    fire(0, 0)

    def pair_body(rr, _):
        r0 = 2 * rr
        fire(r0 + 1, 1)
        drain(0)
        accumulate(0, r0)

        @pl.when(rr < B_PER_W // 2 - 1)
        def _():
            fire(r0 + 2, 0)

        drain(1)
        accumulate(1, r0 + 1)
        return 0

    lax.fori_loop(0, B_PER_W // 2, pair_body, 0)
    pltpu.sync_copy(out_v, out_hbm.at[pl.ds(base, B_PER_W)])


BB = 256


def _embs_sum_body(embs_ref, o_ref):
    o_ref[...] = jnp.sum(embs_ref[...], axis=1)


_embs_sum = pl.pallas_call(
    _embs_sum_body,
    grid=(B // BB,),
    in_specs=[pl.BlockSpec((BB, WL, D), lambda i: (i, 0, 0))],
    out_specs=pl.BlockSpec((BB, D), lambda i: (i, 0)),
    out_shape=jax.ShapeDtypeStruct((B, D), jnp.float32),
)


def _combine_body(es_ref, ng_ref, w1_ref, b1_ref, w2_ref, b2_ref, o_ref):
    x = (es_ref[...] + ng_ref[...]) * (1.0 / (WL + NG))
    h = lax.dot_general(x, w1_ref[...], (((1,), (1,)), ((), ())),
                        preferred_element_type=jnp.float32) + b1_ref[...]
    logits = lax.dot_general(h, w2_ref[...], (((1,), (1,)), ((), ())),
                             preferred_element_type=jnp.float32) + b2_ref[...]
    o_ref[...] = jax.nn.sigmoid(logits)


_combine = pl.pallas_call(
    _combine_body,
    grid=(B // BB,),
    in_specs=[
        pl.BlockSpec((BB, D), lambda i: (i, 0)),
        pl.BlockSpec((BB, D), lambda i: (i, 0)),
        pl.BlockSpec((H, D), lambda i: (0, 0)),
        pl.BlockSpec((1, H), lambda i: (0, 0)),
        pl.BlockSpec((C, H), lambda i: (0, 0)),
        pl.BlockSpec((1, C), lambda i: (0, 0)),
    ],
    out_specs=pl.BlockSpec((BB, C), lambda i: (i, 0)),
    out_shape=jax.ShapeDtypeStruct((B, C), jnp.float32),
)


def kernel(embs, ngram_embs, table, W_i2h, b_i2h, W_h2o, b_h2o):
    idx = ngram_embs.astype(jnp.int32).reshape(-1)
    ng_sum = _ngram_sum(idx, table)
    es = _embs_sum(embs)
    return _combine(es, ng_sum, W_i2h, b_i2h.reshape(1, H),
                    W_h2o, b_h2o.reshape(1, C))


# speedup vs baseline: 2.0621x; 1.2693x over previous
import functools

import jax
import jax.numpy as jnp
from jax import lax
from jax.experimental import pallas as pl
from jax.experimental.pallas import tpu as pltpu
from jax.experimental.pallas import tpu_sc as plsc

B = 4096
D = 128
NG = 200
WL = 50
H = 100
C = 10

NC = 2
NS = 16
NW = NC * NS
B_PER_W = B // NW
LANES = 16

_mesh = plsc.VectorSubcoreMesh(core_axis_name="c", subcore_axis_name="s")


@functools.partial(
    pl.kernel,
    out_type=jax.ShapeDtypeStruct((B, D), jnp.float32),
    mesh=_mesh,
    scratch_types=[
        pltpu.VMEM((B_PER_W, NG), jnp.int32),
        pltpu.VMEM((2, NG, D), jnp.float32),
        pltpu.VMEM((B_PER_W, D), jnp.float32),
        pltpu.SemaphoreType.DMA,
        pltpu.SemaphoreType.DMA,
    ],
)
def _ngram_sum(idx_hbm, table_hbm, out_hbm, idx_v, rows_v, out_v, sem0, sem1):
    wid = lax.axis_index("s") * NC + lax.axis_index("c")
    base = pl.multiple_of(wid * B_PER_W, B_PER_W)
    pltpu.sync_copy(idx_hbm.at[pl.ds(base, B_PER_W)], idx_v)

    sems = (sem0, sem1)

    def fire(r, buf):
        pltpu.async_copy(
            table_hbm.at[idx_v.at[r, pl.ds(0, 128)]],
            rows_v.at[buf].at[pl.ds(0, 128)], sems[buf])
        pltpu.async_copy(
            table_hbm.at[idx_v.at[r, pl.ds(128, NG - 128)]],
            rows_v.at[buf].at[pl.ds(128, NG - 128)], sems[buf])

    def drain(buf):
        pltpu.make_async_copy(
            table_hbm.at[pl.ds(0, NG)], rows_v.at[buf], sems[buf]).wait()

    def accumulate(buf, r):
        def accum(j, accs):
            a = tuple(
                accs[d] + rows_v[buf, 2 * j, pl.ds(d * LANES, LANES)]
                for d in range(D // LANES))
            return tuple(
                a[d] + rows_v[buf, 2 * j + 1, pl.ds(d * LANES, LANES)]
                for d in range(D // LANES))

        accs = lax.fori_loop(
            0, NG // 2, accum,
            tuple(jnp.zeros((LANES,), jnp.float32) for _ in range(D // LANES)))
        for d in range(D // LANES):
            out_v[r, pl.ds(d * LANES, LANES)] = accs[d]

    fire(0, 0)

    def pair_body(rr, _):
        r0 = 2 * rr
        fire(r0 + 1, 1)
        drain(0)
        accumulate(0, r0)

        @pl.when(rr < B_PER_W // 2 - 1)
        def _():
            fire(r0 + 2, 0)

        drain(1)
        accumulate(1, r0 + 1)
        return 0

    lax.fori_loop(0, B_PER_W // 2, pair_body, 0)
    pltpu.sync_copy(out_v, out_hbm.at[pl.ds(base, B_PER_W)])


BB = 256


def _embs_sum_body(embs_ref, o_ref):
    o_ref[...] = jnp.sum(embs_ref[...], axis=0)


_embs_sum = pl.pallas_call(
    _embs_sum_body,
    grid=(B // BB,),
    in_specs=[pl.BlockSpec((WL, BB, D), lambda i: (0, i, 0))],
    out_specs=pl.BlockSpec((BB, D), lambda i: (i, 0)),
    out_shape=jax.ShapeDtypeStruct((B, D), jnp.float32),
)


def _combine_body(es_ref, ng_ref, w1_ref, b1_ref, w2_ref, b2_ref, o_ref):
    x = (es_ref[...] + ng_ref[...]) * (1.0 / (WL + NG))
    h = lax.dot_general(x, w1_ref[...], (((1,), (1,)), ((), ())),
                        preferred_element_type=jnp.float32) + b1_ref[...]
    logits = lax.dot_general(h, w2_ref[...], (((1,), (1,)), ((), ())),
                             preferred_element_type=jnp.float32) + b2_ref[...]
    o_ref[...] = jax.nn.sigmoid(logits)


_combine = pl.pallas_call(
    _combine_body,
    grid=(B // BB,),
    in_specs=[
        pl.BlockSpec((BB, D), lambda i: (i, 0)),
        pl.BlockSpec((BB, D), lambda i: (i, 0)),
        pl.BlockSpec((H, D), lambda i: (0, 0)),
        pl.BlockSpec((1, H), lambda i: (0, 0)),
        pl.BlockSpec((C, H), lambda i: (0, 0)),
        pl.BlockSpec((1, C), lambda i: (0, 0)),
    ],
    out_specs=pl.BlockSpec((BB, C), lambda i: (i, 0)),
    out_shape=jax.ShapeDtypeStruct((B, C), jnp.float32),
)


def kernel(embs, ngram_embs, table, W_i2h, b_i2h, W_h2o, b_h2o):
    idx = ngram_embs.astype(jnp.int32)
    ng_sum = _ngram_sum(idx, table)
    es = _embs_sum(jnp.transpose(embs, (1, 0, 2)))
    return _combine(es, ng_sum, W_i2h, b_i2h.reshape(1, H),
                    W_h2o, b_h2o.reshape(1, C))
